# Initial kernel scaffold; baseline (speedup 1.0000x reference)
import jax
import jax.numpy as jnp
from jax.experimental import pallas as pl


def kernel(q, k, v, u, x_offsets, Wg):
    raise NotImplementedError("write your pallas kernel here")


# speedup vs baseline: 1.1774x; 1.1774x over previous
import jax
import jax.numpy as jnp
from jax.experimental import pallas as pl
from jax.experimental.pallas import tpu as pltpu

B = 4
L = 1024
H = 8
D = 128
BS = 32
NB = L // BS
MQ = 128
NM = L // MQ
SCALE = D ** -0.5
INV_SCALE = 1.0 / SCALE
NEG = -1e30


def _silu(x):
    return x * jax.nn.sigmoid(x)


def _split3(x):
    h1 = x.astype(jnp.bfloat16)
    r1 = x - h1.astype(jnp.float32)
    h2 = r1.astype(jnp.bfloat16)
    r2 = r1 - h2.astype(jnp.float32)
    h3 = r2.astype(jnp.bfloat16)
    return h1, h2, h3


def _dot_t_f32(a, b):
    a1, a2, a3 = _split3(a)
    b1, b2, b3 = _split3(b)
    dims = (((1,), (1,)), ((), ()))

    def dg(x, y):
        return jax.lax.dot_general(x, y, dims,
                                   preferred_element_type=jnp.float32)

    return (dg(a1, b1) + (dg(a1, b2) + dg(a2, b1))
            + (dg(a1, b3) + dg(a3, b1) + dg(a2, b2)))


def _hstu_bsa_kernel(qfull_ref, wg_ref, q_ref, k_ref, v_ref, u_ref,
                     out_ref, g_ref, ocmp_ref, wt_ref):
    h = pl.program_id(1)

    @pl.when(h == 0)
    def _gate():
        g_ref[...] = jax.nn.sigmoid(
            jnp.dot(qfull_ref[0], wg_ref[...],
                    preferred_element_type=jnp.float32))

    Q = q_ref[0]
    K = k_ref[0]
    V = v_ref[0]

    gcol = jax.lax.broadcasted_iota(jnp.int32, (L, 2 * H), 1)
    g = g_ref[...]
    g_cmp = jnp.sum(jnp.where(gcol == h, g, 0.0), axis=1, keepdims=True)

    e_row = jax.lax.broadcasted_iota(jnp.int32, (NB, L), 0)
    e_col = jax.lax.broadcasted_iota(jnp.int32, (NB, L), 1)
    E = (e_col // BS == e_row).astype(jnp.float32)

    k1, k2, k3 = _split3(K)
    Eb = E.astype(jnp.bfloat16)

    def _esum(x1, x2, x3):
        def dg(y):
            return jax.lax.dot_general(
                Eb, y, (((1,), (0,)), ((), ())),
                preferred_element_type=jnp.float32)
        return (dg(x1) + dg(x2) + dg(x3)) * (1.0 / BS)

    kc = _esum(k1, k2, k3)
    vc = jnp.dot(E, V, preferred_element_type=jnp.float32) * (1.0 / BS)

    s_blk = jax.lax.dot_general(
        Q, kc, (((1,), (1,)), ((), ())),
        preferred_element_type=jnp.float32) * SCALE
    l_i = jax.lax.broadcasted_iota(jnp.int32, (L, NB), 0)
    n_i = jax.lax.broadcasted_iota(jnp.int32, (L, NB), 1)
    qblk = l_i // BS
    causal_blk = qblk >= n_i
    p_cmp = jnp.where(causal_blk, _silu(s_blk) * INV_SCALE, 0.0)

    ocmp_ref[...] = jnp.dot(p_cmp, vc,
                            preferred_element_type=jnp.float32) * g_cmp

    sel = jnp.where(qblk == n_i, 1.0, p_cmp)
    work = sel
    idxs = []
    for _ in range(3):
        m = jnp.max(work, axis=1, keepdims=True)
        cand = jnp.where(work == m, n_i, NB)
        it = jnp.min(cand, axis=1, keepdims=True)
        idxs.append(it)
        work = jnp.where(n_i == it, NEG, work)
    i0, _, i2 = idxs
    wt_ref[...] = 2.0 * ((n_i == i0).astype(jnp.float32)
                         + (n_i == i2).astype(jnp.float32))

    for mi in range(NM):
        r0 = mi * MQ
        kl = (mi + 1) * MQ
        Qm = Q[r0:r0 + MQ]
        s = jax.lax.dot_general(
            Qm, K[:kl], (((1,), (1,)), ((), ())),
            preferred_element_type=jnp.float32) * SCALE
        row = jax.lax.broadcasted_iota(jnp.int32, (MQ, kl), 0) + r0
        col = jax.lax.broadcasted_iota(jnp.int32, (MQ, kl), 1)
        p = jnp.where(row >= col, _silu(s) * INV_SCALE, 0.0)
        w_tok = jnp.dot(wt_ref[r0:r0 + MQ, :], E[:, :kl],
                        preferred_element_type=jnp.float32)
        o_slc = jnp.dot(p * w_tok, V[:kl],
                        preferred_element_type=jnp.float32)
        gg = g_ref[r0:r0 + MQ, :]
        gcol_m = jax.lax.broadcasted_iota(jnp.int32, (MQ, 2 * H), 1)
        g_slc_m = jnp.sum(jnp.where(gcol_m == h + H, gg, 0.0),
                          axis=1, keepdims=True)
        out_ref[0, r0:r0 + MQ, :] = u_ref[0, r0:r0 + MQ, :] * (
            ocmp_ref[r0:r0 + MQ, :] + g_slc_m * o_slc)


def kernel(q, k, v, u, x_offsets, Wg):
    del x_offsets
    qf = q.reshape(B, L, H * D)
    kf = k.reshape(B, L, H * D)
    vf = v.reshape(B, L, H * D)
    uf = u.reshape(B, L, H * D)

    head = pl.BlockSpec((1, L, D), lambda b, h: (b, 0, h))
    out = pl.pallas_call(
        _hstu_bsa_kernel,
        grid=(B, H),
        in_specs=[
            pl.BlockSpec((1, L, H * D), lambda b, h: (b, 0, 0)),
            pl.BlockSpec((H * D, 2 * H), lambda b, h: (0, 0)),
            head, head, head, head,
        ],
        out_specs=pl.BlockSpec((1, L, D), lambda b, h: (b * H + h, 0, 0)),
        out_shape=jax.ShapeDtypeStruct((B * H, L, D), jnp.float32),
        scratch_shapes=[pltpu.VMEM((L, 2 * H), jnp.float32),
                        pltpu.VMEM((L, D), jnp.float32),
                        pltpu.VMEM((L, NB), jnp.float32)],
    )(qf, Wg, qf, kf, vf, uf)
    return out.reshape(B, H, L, D).transpose(0, 2, 1, 3).reshape(B * L, H, D)


# speedup vs baseline: 1.1895x; 1.0103x over previous
import jax
import jax.numpy as jnp
from jax.experimental import pallas as pl
from jax.experimental.pallas import tpu as pltpu

B = 4
L = 1024
H = 8
D = 128
BS = 32
NB = L // BS
MQ = 128
NM = L // MQ
SCALE = D ** -0.5
INV_SCALE = 1.0 / SCALE
NEG = -1e30


def _silu(x):
    return x * jax.nn.sigmoid(x)


def _split3(x):
    h1 = x.astype(jnp.bfloat16)
    r1 = x - h1.astype(jnp.float32)
    h2 = r1.astype(jnp.bfloat16)
    r2 = r1 - h2.astype(jnp.float32)
    h3 = r2.astype(jnp.bfloat16)
    return h1, h2, h3


def _hstu_bsa_kernel(q_ref, k_ref, v_ref, u_ref, wg_ref, out_ref, g_ref):
    g_ref[...] = jax.nn.sigmoid(
        jnp.dot(q_ref[0], wg_ref[...], preferred_element_type=jnp.float32))

    e_row = jax.lax.broadcasted_iota(jnp.int32, (NB, L), 0)
    e_col = jax.lax.broadcasted_iota(jnp.int32, (NB, L), 1)
    E = (e_col // BS == e_row).astype(jnp.float32)
    Eb = E.astype(jnp.bfloat16)

    l_i = jax.lax.broadcasted_iota(jnp.int32, (L, NB), 0)
    n_i = jax.lax.broadcasted_iota(jnp.int32, (L, NB), 1)
    qblk = l_i // BS
    causal_blk = qblk >= n_i
    diag_blk = qblk == n_i
    gcol = jax.lax.broadcasted_iota(jnp.int32, (L, 2 * H), 1)

    for h in range(H):
        c0 = h * D
        Q = q_ref[0, :, c0:c0 + D]
        K = k_ref[0, :, c0:c0 + D]
        V = v_ref[0, :, c0:c0 + D]

        g = g_ref[...]
        g_cmp = jnp.sum(jnp.where(gcol == h, g, 0.0), axis=1, keepdims=True)
        g_slc = jnp.sum(jnp.where(gcol == h + H, g, 0.0),
                        axis=1, keepdims=True)

        k1, k2, k3 = _split3(K)

        def dg0(y):
            return jax.lax.dot_general(
                Eb, y, (((1,), (0,)), ((), ())),
                preferred_element_type=jnp.float32)

        kc = (dg0(k1) + dg0(k2) + dg0(k3)) * (1.0 / BS)
        vc = jnp.dot(E, V, preferred_element_type=jnp.float32) * (1.0 / BS)

        s_blk = jax.lax.dot_general(
            Q, kc, (((1,), (1,)), ((), ())),
            preferred_element_type=jnp.float32) * SCALE
        p_cmp = jnp.where(causal_blk, _silu(s_blk) * INV_SCALE, 0.0)

        o_cmp = jnp.dot(p_cmp, vc,
                        preferred_element_type=jnp.float32) * g_cmp

        work = jnp.where(diag_blk, 1.0, p_cmp)
        idxs = []
        for _ in range(3):
            m = jnp.max(work, axis=1, keepdims=True)
            cand = jnp.where(work == m, n_i, NB)
            it = jnp.min(cand, axis=1, keepdims=True)
            idxs.append(it)
            work = jnp.where(n_i == it, NEG, work)
        i0, _, i2 = idxs
        wt = 2.0 * ((n_i == i0).astype(jnp.float32)
                    + (n_i == i2).astype(jnp.float32))

        for mi in range(NM):
            r0 = mi * MQ
            kl = (mi + 1) * MQ
            s = jax.lax.dot_general(
                Q[r0:r0 + MQ], K[:kl], (((1,), (1,)), ((), ())),
                preferred_element_type=jnp.float32) * SCALE
            row = jax.lax.broadcasted_iota(jnp.int32, (MQ, kl), 0) + r0
            col = jax.lax.broadcasted_iota(jnp.int32, (MQ, kl), 1)
            p = jnp.where(row >= col, _silu(s) * INV_SCALE, 0.0)
            w_tok = jnp.dot(wt[r0:r0 + MQ], E[:, :kl],
                            preferred_element_type=jnp.float32)
            o_slc = jnp.dot(p * w_tok, V[:kl],
                            preferred_element_type=jnp.float32)
            out_ref[0, r0:r0 + MQ, c0:c0 + D] = u_ref[0, r0:r0 + MQ,
                                                      c0:c0 + D] * (
                o_cmp[r0:r0 + MQ] + g_slc[r0:r0 + MQ] * o_slc)


def kernel(q, k, v, u, x_offsets, Wg):
    del x_offsets
    qf = q.reshape(B, L, H * D)
    kf = k.reshape(B, L, H * D)
    vf = v.reshape(B, L, H * D)
    uf = u.reshape(B, L, H * D)

    slab = pl.BlockSpec((1, L, H * D), lambda b: (b, 0, 0))
    out = pl.pallas_call(
        _hstu_bsa_kernel,
        grid=(B,),
        in_specs=[slab, slab, slab, slab,
                  pl.BlockSpec((H * D, 2 * H), lambda b: (0, 0))],
        out_specs=slab,
        out_shape=jax.ShapeDtypeStruct((B, L, H * D), jnp.float32),
        scratch_shapes=[pltpu.VMEM((L, 2 * H), jnp.float32)],
    )(qf, kf, vf, uf, Wg)
    return out.reshape(B * L, H, D)


# speedup vs baseline: 1.3613x; 1.1444x over previous
import jax
import jax.numpy as jnp
from jax.experimental import pallas as pl
from jax.experimental.pallas import tpu as pltpu

B = 4
L = 1024
H = 8
D = 128
BS = 32
NB = L // BS
MQ = 128
NM = L // MQ
SCALE = D ** -0.5
INV_SCALE = 1.0 / SCALE
NEG = -1e30


def _silu(x):
    return x * jax.nn.sigmoid(x)


def _split3(x):
    h1 = x.astype(jnp.bfloat16)
    r1 = x - h1.astype(jnp.float32)
    h2 = r1.astype(jnp.bfloat16)
    r2 = r1 - h2.astype(jnp.float32)
    h3 = r2.astype(jnp.bfloat16)
    return h1, h2, h3


def _dg(a, b, dims):
    return jax.lax.dot_general(a, b, (dims, ((), ())),
                               preferred_element_type=jnp.float32)


def _hstu_bsa_kernel(q_ref, k_ref, v_ref, u_ref, wg_ref, out_ref, g_ref):
    g_ref[...] = jax.nn.sigmoid(
        jnp.dot(q_ref[0], wg_ref[...], preferred_element_type=jnp.float32))

    e_row = jax.lax.broadcasted_iota(jnp.int32, (NB, L), 0)
    e_col = jax.lax.broadcasted_iota(jnp.int32, (NB, L), 1)
    E = (e_col // BS == e_row).astype(jnp.float32)
    Eb = E.astype(jnp.bfloat16)
    t_row = jax.lax.broadcasted_iota(jnp.int32, (L, NB), 0)
    t_col = jax.lax.broadcasted_iota(jnp.int32, (L, NB), 1)
    Et = (t_row // BS == t_col).astype(jnp.bfloat16)

    n_i = e_row
    qblk = e_col // BS
    causal_blk = qblk >= n_i
    diag_blk = qblk == n_i
    gcol = jax.lax.broadcasted_iota(jnp.int32, (L, 2 * H), 1)

    d_key = jax.lax.broadcasted_iota(jnp.int32, (MQ, MQ), 0)
    d_qry = jax.lax.broadcasted_iota(jnp.int32, (MQ, MQ), 1)
    diag_keep = d_key <= d_qry

    for h in range(H):
        c0 = h * D
        Q = q_ref[0, :, c0:c0 + D]
        K = k_ref[0, :, c0:c0 + D]
        V = v_ref[0, :, c0:c0 + D]

        g = g_ref[...]
        g_cmp = jnp.sum(jnp.where(gcol == h, g, 0.0), axis=1, keepdims=True)
        g_slc = jnp.sum(jnp.where(gcol == h + H, g, 0.0),
                        axis=1, keepdims=True)

        k1, k2, k3 = _split3(K)
        kc = (_dg(Eb, k1, ((1,), (0,))) + _dg(Eb, k2, ((1,), (0,)))
              + _dg(Eb, k3, ((1,), (0,)))) * (1.0 / BS)
        vc = _dg(E, V, ((1,), (0,))) * (1.0 / BS)

        s_blk = _dg(kc, Q, ((1,), (1,))) * SCALE
        p_cmp = jnp.where(causal_blk, _silu(s_blk) * INV_SCALE, 0.0)

        o_cmp = _dg(p_cmp, vc, ((0,), (0,))) * g_cmp

        work = jnp.where(diag_blk, 1.0, p_cmp)
        idxs = []
        for _ in range(3):
            m = jnp.max(work, axis=0, keepdims=True)
            cand = jnp.where(work == m, n_i, NB)
            it = jnp.min(cand, axis=0, keepdims=True)
            idxs.append(it)
            work = jnp.where(n_i == it, NEG, work)
        i0, _, i2 = idxs
        wt = 2.0 * ((n_i == i0).astype(jnp.float32)
                    + (n_i == i2).astype(jnp.float32))

        for mi in range(NM):
            r0 = mi * MQ
            Qm = Q[r0:r0 + MQ]
            wtm = wt[:, r0:r0 + MQ]

            s_d = _dg(K[r0:r0 + MQ], Qm, ((1,), (1,))) * SCALE
            w_d = _dg(Et[r0:r0 + MQ], wtm, ((1,), (0,)))
            pw_d = jnp.where(diag_keep, _silu(s_d) * INV_SCALE, 0.0) * w_d
            o_slc = _dg(pw_d, V[r0:r0 + MQ], ((0,), (0,)))

            if r0 > 0:
                s_t = _dg(K[:r0], Qm, ((1,), (1,))) * SCALE
                w_t = _dg(Et[:r0], wtm, ((1,), (0,)))
                pw_t = _silu(s_t) * INV_SCALE * w_t
                o_slc = o_slc + _dg(pw_t, V[:r0], ((0,), (0,)))

            out_ref[0, r0:r0 + MQ, c0:c0 + D] = u_ref[0, r0:r0 + MQ,
                                                      c0:c0 + D] * (
                o_cmp[r0:r0 + MQ] + g_slc[r0:r0 + MQ] * o_slc)


def kernel(q, k, v, u, x_offsets, Wg):
    del x_offsets
    qf = q.reshape(B, L, H * D)
    kf = k.reshape(B, L, H * D)
    vf = v.reshape(B, L, H * D)
    uf = u.reshape(B, L, H * D)

    slab = pl.BlockSpec((1, L, H * D), lambda b: (b, 0, 0))
    out = pl.pallas_call(
        _hstu_bsa_kernel,
        grid=(B,),
        in_specs=[slab, slab, slab, slab,
                  pl.BlockSpec((H * D, 2 * H), lambda b: (0, 0))],
        out_specs=slab,
        out_shape=jax.ShapeDtypeStruct((B, L, H * D), jnp.float32),
        scratch_shapes=[pltpu.VMEM((L, 2 * H), jnp.float32)],
    )(qf, kf, vf, uf, Wg)
    return out.reshape(B * L, H, D)


# speedup vs baseline: 1.5860x; 1.1651x over previous
import jax
import jax.numpy as jnp
from jax.experimental import pallas as pl
from jax.experimental.pallas import tpu as pltpu

B = 4
L = 1024
H = 8
D = 128
BS = 32
NB = L // BS
MQ = 256
NM = L // MQ
SCALE = D ** -0.5
INV_SCALE = 1.0 / SCALE
NEG = -1e30


def _silu(x):
    return x * jax.nn.sigmoid(x)


def _split3(x):
    h1 = x.astype(jnp.bfloat16)
    r1 = x - h1.astype(jnp.float32)
    h2 = r1.astype(jnp.bfloat16)
    r2 = r1 - h2.astype(jnp.float32)
    h3 = r2.astype(jnp.bfloat16)
    return h1, h2, h3


def _dg(a, b, dims):
    return jax.lax.dot_general(a, b, (dims, ((), ())),
                               preferred_element_type=jnp.float32)


def _hstu_bsa_kernel(q_ref, k_ref, v_ref, u_ref, wg_ref, out_ref, g_ref):
    g_ref[...] = jax.nn.sigmoid(
        jnp.dot(q_ref[0], wg_ref[...], preferred_element_type=jnp.float32))

    e_row = jax.lax.broadcasted_iota(jnp.int32, (NB, L), 0)
    e_col = jax.lax.broadcasted_iota(jnp.int32, (NB, L), 1)
    E = (e_col // BS == e_row).astype(jnp.float32)
    Eb = E.astype(jnp.bfloat16)
    t_row = jax.lax.broadcasted_iota(jnp.int32, (L, NB), 0)
    t_col = jax.lax.broadcasted_iota(jnp.int32, (L, NB), 1)
    Et = (t_row // BS == t_col).astype(jnp.bfloat16)

    n_i = e_row
    qblk = e_col // BS
    causal_blk = qblk >= n_i
    diag_blk = qblk == n_i
    gcol = jax.lax.broadcasted_iota(jnp.int32, (L, 2 * H), 1)

    d_key = jax.lax.broadcasted_iota(jnp.int32, (MQ, MQ), 0)
    d_qry = jax.lax.broadcasted_iota(jnp.int32, (MQ, MQ), 1)
    diag_keep = d_key <= d_qry

    for h in range(H):
        c0 = h * D
        Q = q_ref[0, :, c0:c0 + D]
        K = k_ref[0, :, c0:c0 + D]
        V = v_ref[0, :, c0:c0 + D]

        g = g_ref[...]
        g_cmp = jnp.sum(jnp.where(gcol == h, g, 0.0), axis=1, keepdims=True)
        g_slc = jnp.sum(jnp.where(gcol == h + H, g, 0.0),
                        axis=1, keepdims=True)

        k1, k2, k3 = _split3(K)
        kc = (_dg(Eb, k1, ((1,), (0,))) + _dg(Eb, k2, ((1,), (0,)))
              + _dg(Eb, k3, ((1,), (0,)))) * (1.0 / BS)
        vc = _dg(E, V, ((1,), (0,))) * (1.0 / BS)

        s_blk = _dg(kc, Q, ((1,), (1,))) * SCALE
        p_cmp = jnp.where(causal_blk, _silu(s_blk) * INV_SCALE, 0.0)

        o_cmp = _dg(p_cmp, vc, ((0,), (0,))) * g_cmp

        work = jnp.where(diag_blk, 1.0, p_cmp)
        idxs = []
        for _ in range(3):
            m = jnp.max(work, axis=0, keepdims=True)
            cand = jnp.where(work == m, n_i, NB)
            it = jnp.min(cand, axis=0, keepdims=True)
            idxs.append(it)
            work = jnp.where(n_i == it, NEG, work)
        i0, _, i2 = idxs
        wt = 2.0 * ((n_i == i0).astype(jnp.float32)
                    + (n_i == i2).astype(jnp.float32))

        for mi in range(NM):
            r0 = mi * MQ
            Qm = Q[r0:r0 + MQ]
            wtm = wt[:, r0:r0 + MQ]

            s_d = _dg(K[r0:r0 + MQ], Qm, ((1,), (1,))) * SCALE
            w_d = _dg(Et[r0:r0 + MQ], wtm, ((1,), (0,)))
            pw_d = jnp.where(diag_keep, _silu(s_d) * INV_SCALE, 0.0) * w_d
            o_slc = _dg(pw_d, V[r0:r0 + MQ], ((0,), (0,)))

            if r0 > 0:
                s_t = _dg(K[:r0], Qm, ((1,), (1,))) * SCALE
                w_t = _dg(Et[:r0], wtm, ((1,), (0,)))
                pw_t = _silu(s_t) * INV_SCALE * w_t
                o_slc = o_slc + _dg(pw_t, V[:r0], ((0,), (0,)))

            out_ref[0, r0:r0 + MQ, c0:c0 + D] = u_ref[0, r0:r0 + MQ,
                                                      c0:c0 + D] * (
                o_cmp[r0:r0 + MQ] + g_slc[r0:r0 + MQ] * o_slc)


def kernel(q, k, v, u, x_offsets, Wg):
    del x_offsets
    qf = q.reshape(B, L, H * D)
    kf = k.reshape(B, L, H * D)
    vf = v.reshape(B, L, H * D)
    uf = u.reshape(B, L, H * D)

    slab = pl.BlockSpec((1, L, H * D), lambda b: (b, 0, 0))
    out = pl.pallas_call(
        _hstu_bsa_kernel,
        grid=(B,),
        in_specs=[slab, slab, slab, slab,
                  pl.BlockSpec((H * D, 2 * H), lambda b: (0, 0))],
        out_specs=slab,
        out_shape=jax.ShapeDtypeStruct((B, L, H * D), jnp.float32),
        scratch_shapes=[pltpu.VMEM((L, 2 * H), jnp.float32)],
    )(qf, kf, vf, uf, Wg)
    return out.reshape(B * L, H, D)


# speedup vs baseline: 1.6502x; 1.0404x over previous
import jax
import jax.numpy as jnp
from jax.experimental import pallas as pl
from jax.experimental.pallas import tpu as pltpu

B = 4
L = 1024
H = 8
D = 128
BS = 32
NB = L // BS
MQ = 512
NM = L // MQ
SCALE = D ** -0.5
INV_SCALE = 1.0 / SCALE
NEG = -1e30


def _silu(x):
    return x * jax.nn.sigmoid(x)


def _split3(x):
    h1 = x.astype(jnp.bfloat16)
    r1 = x - h1.astype(jnp.float32)
    h2 = r1.astype(jnp.bfloat16)
    r2 = r1 - h2.astype(jnp.float32)
    h3 = r2.astype(jnp.bfloat16)
    return h1, h2, h3


def _dg(a, b, dims):
    return jax.lax.dot_general(a, b, (dims, ((), ())),
                               preferred_element_type=jnp.float32)


def _hstu_bsa_kernel(q_ref, k_ref, v_ref, u_ref, wg_ref, out_ref, g_ref):
    g_ref[...] = jax.nn.sigmoid(
        jnp.dot(q_ref[0], wg_ref[...], preferred_element_type=jnp.float32))

    e_row = jax.lax.broadcasted_iota(jnp.int32, (NB, L), 0)
    e_col = jax.lax.broadcasted_iota(jnp.int32, (NB, L), 1)
    E = (e_col // BS == e_row).astype(jnp.float32)
    Eb = E.astype(jnp.bfloat16)
    t_row = jax.lax.broadcasted_iota(jnp.int32, (L, NB), 0)
    t_col = jax.lax.broadcasted_iota(jnp.int32, (L, NB), 1)
    Et = (t_row // BS == t_col).astype(jnp.bfloat16)

    n_i = e_row
    qblk = e_col // BS
    causal_blk = qblk >= n_i
    diag_blk = qblk == n_i
    gcol = jax.lax.broadcasted_iota(jnp.int32, (L, 2 * H), 1)

    d_key = jax.lax.broadcasted_iota(jnp.int32, (MQ, MQ), 0)
    d_qry = jax.lax.broadcasted_iota(jnp.int32, (MQ, MQ), 1)
    diag_keep = d_key <= d_qry

    for h in range(H):
        c0 = h * D
        Q = q_ref[0, :, c0:c0 + D]
        K = k_ref[0, :, c0:c0 + D]
        V = v_ref[0, :, c0:c0 + D]

        g = g_ref[...]
        g_cmp = jnp.sum(jnp.where(gcol == h, g, 0.0), axis=1, keepdims=True)
        g_slc = jnp.sum(jnp.where(gcol == h + H, g, 0.0),
                        axis=1, keepdims=True)

        k1, k2, k3 = _split3(K)
        kc = (_dg(Eb, k1, ((1,), (0,))) + _dg(Eb, k2, ((1,), (0,)))
              + _dg(Eb, k3, ((1,), (0,)))) * (1.0 / BS)
        vc = _dg(E, V, ((1,), (0,))) * (1.0 / BS)

        s_blk = _dg(kc, Q, ((1,), (1,))) * SCALE
        p_cmp = jnp.where(causal_blk, _silu(s_blk) * INV_SCALE, 0.0)

        o_cmp = _dg(p_cmp, vc, ((0,), (0,))) * g_cmp

        work = jnp.where(diag_blk, 1.0, p_cmp)
        idxs = []
        for _ in range(3):
            m = jnp.max(work, axis=0, keepdims=True)
            cand = jnp.where(work == m, n_i, NB)
            it = jnp.min(cand, axis=0, keepdims=True)
            idxs.append(it)
            work = jnp.where(n_i == it, NEG, work)
        i0, _, i2 = idxs
        wt = 2.0 * ((n_i == i0).astype(jnp.float32)
                    + (n_i == i2).astype(jnp.float32))

        for mi in range(NM):
            r0 = mi * MQ
            Qm = Q[r0:r0 + MQ]
            wtm = wt[:, r0:r0 + MQ]

            s_d = _dg(K[r0:r0 + MQ], Qm, ((1,), (1,))) * SCALE
            w_d = _dg(Et[r0:r0 + MQ], wtm, ((1,), (0,)))
            pw_d = jnp.where(diag_keep, _silu(s_d) * INV_SCALE, 0.0) * w_d
            o_slc = _dg(pw_d, V[r0:r0 + MQ], ((0,), (0,)))

            if r0 > 0:
                s_t = _dg(K[:r0], Qm, ((1,), (1,))) * SCALE
                w_t = _dg(Et[:r0], wtm, ((1,), (0,)))
                pw_t = _silu(s_t) * INV_SCALE * w_t
                o_slc = o_slc + _dg(pw_t, V[:r0], ((0,), (0,)))

            out_ref[0, r0:r0 + MQ, c0:c0 + D] = u_ref[0, r0:r0 + MQ,
                                                      c0:c0 + D] * (
                o_cmp[r0:r0 + MQ] + g_slc[r0:r0 + MQ] * o_slc)


def kernel(q, k, v, u, x_offsets, Wg):
    del x_offsets
    qf = q.reshape(B, L, H * D)
    kf = k.reshape(B, L, H * D)
    vf = v.reshape(B, L, H * D)
    uf = u.reshape(B, L, H * D)

    slab = pl.BlockSpec((1, L, H * D), lambda b: (b, 0, 0))
    out = pl.pallas_call(
        _hstu_bsa_kernel,
        grid=(B,),
        in_specs=[slab, slab, slab, slab,
                  pl.BlockSpec((H * D, 2 * H), lambda b: (0, 0))],
        out_specs=slab,
        out_shape=jax.ShapeDtypeStruct((B, L, H * D), jnp.float32),
        scratch_shapes=[pltpu.VMEM((L, 2 * H), jnp.float32)],
    )(qf, kf, vf, uf, Wg)
    return out.reshape(B * L, H, D)
